# Initial kernel scaffold; baseline (speedup 1.0000x reference)
import jax
import jax.numpy as jnp
from jax.experimental import pallas as pl


def kernel(pred_points, gt_points):
    raise NotImplementedError("write your pallas kernel here")


# speedup vs baseline: 1.5925x; 1.5925x over previous
import functools

import jax
import jax.numpy as jnp
from jax.experimental import pallas as pl
from jax.experimental.pallas import tpu as pltpu

B, N, M = 8, 2048, 2048
TN = 256
NI = N // TN


def _chamfer_tc_body(pxt_ref, pyt_ref, gx_ref, gy_ref, acc_ref, gtmin_ref):
    i = pl.program_id(0)

    psum = jnp.float32(0.0)
    for b in range(B):
        pxc = pxt_ref[:, b : b + 1]
        pyc = pyt_ref[:, b : b + 1]
        gxr = gx_ref[b : b + 1, :]
        gyr = gy_ref[b : b + 1, :]
        dx = pxc - gxr
        dy = pyc - gyr
        dist = dx * dx + dy * dy
        psum = psum + jnp.sum(jnp.min(dist, axis=1))
        gt_part = jnp.min(dist, axis=0, keepdims=True)

        @pl.when(i == 0)
        def _init_gt():
            gtmin_ref[b : b + 1, :] = gt_part

        @pl.when(i != 0)
        def _acc_gt():
            gtmin_ref[b : b + 1, :] = jnp.minimum(gtmin_ref[b : b + 1, :], gt_part)

    @pl.when(i == 0)
    def _init_acc():
        acc_ref[0, 0] = 0.0

    acc_ref[0, 0] += psum * (1.0 / (B * N))

    @pl.when(i == NI - 1)
    def _flush_gt():
        acc_ref[0, 0] += jnp.sum(gtmin_ref[:, :]) * (1.0 / (B * M))


@functools.partial(jax.jit, static_argnames=("interpret",))
def _chamfer_tc(pxt, pyt, gx, gy, interpret=False):
    out = pl.pallas_call(
        _chamfer_tc_body,
        grid=(NI,),
        in_specs=[
            pl.BlockSpec((TN, B), lambda i: (i, 0)),
            pl.BlockSpec((TN, B), lambda i: (i, 0)),
            pl.BlockSpec((B, M), lambda i: (0, 0)),
            pl.BlockSpec((B, M), lambda i: (0, 0)),
        ],
        out_specs=pl.BlockSpec((1, 1), lambda i: (0, 0), memory_space=pltpu.SMEM),
        out_shape=jax.ShapeDtypeStruct((1, 1), jnp.float32),
        scratch_shapes=[pltpu.VMEM((B, M), jnp.float32)],
        interpret=interpret,
    )(pxt, pyt, gx, gy)
    return out[0, 0]


def kernel(pred_points, gt_points, interpret=False):
    pxt = pred_points[:, :, 0].T
    pyt = pred_points[:, :, 1].T
    gx = gt_points[:, :, 0]
    gy = gt_points[:, :, 1]
    return _chamfer_tc(pxt, pyt, gx, gy, interpret=interpret)
